# baseline (speedup 1.0000x reference)
import functools

import jax
import jax.numpy as jnp
from jax import lax
from jax.experimental import pallas as pl
from jax.experimental.pallas import tpu as pltpu
from jax.experimental.pallas import tpu_sc as plsc

ROWS = 128
COLS = 32768
LANES = 16
NUM_CORES = 2
NUM_SUBCORES = 16
ROWS_PER_WORKER = ROWS // (NUM_CORES * NUM_SUBCORES)
ROWS_PER_CORE = ROWS // NUM_CORES
VECS = COLS // LANES
UNROLL = 16
NACC = 4
BLOCK = LANES * UNROLL
NBLOCKS = COLS // BLOCK


def _row_argmin(buf, lane_iota):
    def body(i, carry):
        minvs, blks = carry
        minvs = list(minvs)
        blks = list(blks)
        base = i * BLOCK
        ivec = jnp.full((LANES,), 0, jnp.int32) + i
        for u in range(UNROLL):
            k = u % NACC
            v = buf[pl.ds(base + u * LANES, LANES)]
            pred = v < minvs[k]
            minvs[k] = jnp.where(pred, v, minvs[k])
            blks[k] = jnp.where(pred, ivec, blks[k])
        return tuple(minvs), tuple(blks)

    minv0 = jnp.full((LANES,), jnp.inf, jnp.float32)
    blk0 = jnp.zeros((LANES,), jnp.int32)
    minvs, blks = lax.fori_loop(
        0, NBLOCKS, body, ((minv0,) * NACC, (blk0,) * NACC)
    )
    m = jnp.min(jnp.minimum(
        jnp.minimum(minvs[0], minvs[1]), jnp.minimum(minvs[2], minvs[3])
    ))
    bigb = jnp.int32(NBLOCKS)
    cb = jnp.minimum(
        jnp.minimum(
            jnp.where(minvs[0] == m, blks[0], bigb),
            jnp.where(minvs[1] == m, blks[1], bigb),
        ),
        jnp.minimum(
            jnp.where(minvs[2] == m, blks[2], bigb),
            jnp.where(minvs[3] == m, blks[3], bigb),
        ),
    )
    b = jnp.min(cb)
    base = b * BLOCK
    mini = jnp.full((LANES,), COLS, jnp.int32)
    for u in range(UNROLL):
        v = buf[pl.ds(base + u * LANES, LANES)]
        idxv = lane_iota + (base + u * LANES)
        mini = jnp.minimum(mini, jnp.where(v == m, idxv, jnp.int32(COLS)))
    return jnp.min(mini)


@functools.partial(
    pl.kernel,
    out_type=(
        jax.ShapeDtypeStruct((ROWS, 1), jnp.int32),
        jax.ShapeDtypeStruct((ROWS,), jnp.int32),
    ),
    mesh=plsc.VectorSubcoreMesh(
        core_axis_name="c",
        subcore_axis_name="s",
        num_cores=NUM_CORES,
        num_subcores=NUM_SUBCORES,
    ),
    scratch_types=[
        pltpu.VMEM((COLS,), jnp.float32),
        pltpu.VMEM((COLS,), jnp.float32),
        pltpu.VMEM((LANES,), jnp.int32),
        pltpu.VMEM((NUM_SUBCORES, LANES), jnp.int32),
        pltpu.VMEM((ROWS_PER_CORE,), jnp.int32),
        pltpu.VMEM((ROWS_PER_CORE, 1), jnp.int32),
        pltpu.VMEM_SHARED((NUM_SUBCORES, LANES), jnp.int32),
        pltpu.SemaphoreType.DMA,
        pltpu.SemaphoreType.DMA,
    ],
    compiler_params=pltpu.CompilerParams(
        needs_layout_passes=False, use_tc_tiling_on_sc=False
    ),
)
def _argmin_sc(x_hbm, out_kd, out_flat, buf0, buf1, res_ref, gbuf, obuf,
               obuf2, shared, sem0, sem1):
    c = lax.axis_index("c")
    s = lax.axis_index("s")
    base_row = c * ROWS_PER_CORE + s * ROWS_PER_WORKER
    lane_iota = lax.iota(jnp.int32, LANES)
    bufs = (buf0, buf1)
    sems = (sem0, sem1)

    copies = [None] * ROWS_PER_WORKER
    copies[0] = pltpu.async_copy(x_hbm.at[base_row], buf0, sem0)
    res = jnp.zeros((LANES,), jnp.int32)
    for j in range(ROWS_PER_WORKER):
        copies[j].wait()
        if j + 1 < ROWS_PER_WORKER:
            copies[j + 1] = pltpu.async_copy(
                x_hbm.at[base_row + j + 1], bufs[(j + 1) % 2], sems[(j + 1) % 2]
            )
        val = _row_argmin(bufs[j % 2], lane_iota)
        res = jnp.where(lane_iota == j, val, res)
    res_ref[...] = res
    pltpu.sync_copy(res_ref, shared.at[s])
    plsc.subcore_barrier()

    @pl.when(s == 0)
    def _():
        pltpu.sync_copy(shared, gbuf)
        zeros = jnp.zeros((LANES,), jnp.int32)
        for v in range(ROWS_PER_CORE // LANES):
            rp = lane_iota + v * LANES
            g = plsc.load_gather(gbuf, [rp >> 2, rp & 3])
            obuf[pl.ds(v * LANES, LANES)] = g
            plsc.store_scatter(obuf2, [rp, zeros], g)
        pltpu.sync_copy(obuf, out_flat.at[pl.ds(c * ROWS_PER_CORE, ROWS_PER_CORE)])
        pltpu.sync_copy(
            obuf2,
            out_kd.at[pl.ds(c * ROWS_PER_CORE, ROWS_PER_CORE), pl.ds(0, 1)],
        )


def kernel(x):
    out_kd, out_flat = _argmin_sc(x)
    return (out_kd, out_flat)


# speedup vs baseline: 1.5172x; 1.5172x over previous
import functools

import jax
import jax.numpy as jnp
from jax import lax
from jax.experimental import pallas as pl
from jax.experimental.pallas import tpu as pltpu
from jax.experimental.pallas import tpu_sc as plsc

ROWS = 128
COLS = 32768
LANES = 16
NUM_CORES = 2
NUM_SUBCORES = 16
ROWS_PER_WORKER = ROWS // (NUM_CORES * NUM_SUBCORES)
ROWS_PER_CORE = ROWS // NUM_CORES
VECS = COLS // LANES
UNROLL = 16
NACC = 4
BLOCK = LANES * UNROLL
NBLOCKS = COLS // BLOCK


def _row_argmin(buf, lane_iota):
    def body(i, carry):
        minvs, blks = carry
        minvs = list(minvs)
        blks = list(blks)
        base = i * BLOCK
        ivec = jnp.full((LANES,), 0, jnp.int32) + i
        for u in range(UNROLL):
            k = u % NACC
            v = buf[pl.ds(base + u * LANES, LANES)]
            pred = v < minvs[k]
            minvs[k] = jnp.where(pred, v, minvs[k])
            blks[k] = jnp.where(pred, ivec, blks[k])
        return tuple(minvs), tuple(blks)

    minv0 = jnp.full((LANES,), jnp.inf, jnp.float32)
    blk0 = jnp.zeros((LANES,), jnp.int32)
    minvs, blks = lax.fori_loop(
        0, NBLOCKS, body, ((minv0,) * NACC, (blk0,) * NACC)
    )
    m = jnp.min(jnp.minimum(
        jnp.minimum(minvs[0], minvs[1]), jnp.minimum(minvs[2], minvs[3])
    ))
    bigb = jnp.int32(NBLOCKS)
    cb = jnp.minimum(
        jnp.minimum(
            jnp.where(minvs[0] == m, blks[0], bigb),
            jnp.where(minvs[1] == m, blks[1], bigb),
        ),
        jnp.minimum(
            jnp.where(minvs[2] == m, blks[2], bigb),
            jnp.where(minvs[3] == m, blks[3], bigb),
        ),
    )
    b = jnp.min(cb)
    base = b * BLOCK
    mini = jnp.full((LANES,), COLS, jnp.int32)
    for u in range(UNROLL):
        v = buf[pl.ds(base + u * LANES, LANES)]
        idxv = lane_iota + (base + u * LANES)
        mini = jnp.minimum(mini, jnp.where(v == m, idxv, jnp.int32(COLS)))
    return jnp.min(mini)


@functools.partial(
    pl.kernel,
    out_type=jax.ShapeDtypeStruct((ROWS,), jnp.int32),
    mesh=plsc.VectorSubcoreMesh(
        core_axis_name="c",
        subcore_axis_name="s",
        num_cores=NUM_CORES,
        num_subcores=NUM_SUBCORES,
    ),
    scratch_types=[
        pltpu.VMEM((COLS,), jnp.float32),
        pltpu.VMEM((COLS,), jnp.float32),
        pltpu.VMEM((LANES,), jnp.int32),
        pltpu.VMEM((NUM_SUBCORES, LANES), jnp.int32),
        pltpu.VMEM((ROWS_PER_CORE,), jnp.int32),
        pltpu.VMEM_SHARED((NUM_SUBCORES, LANES), jnp.int32),
        pltpu.SemaphoreType.DMA,
        pltpu.SemaphoreType.DMA,
    ],
    compiler_params=pltpu.CompilerParams(needs_layout_passes=False),
)
def _argmin_sc(x_hbm, out_flat, buf0, buf1, res_ref, gbuf, obuf,
               shared, sem0, sem1):
    c = lax.axis_index("c")
    s = lax.axis_index("s")
    base_row = c * ROWS_PER_CORE + s * ROWS_PER_WORKER
    lane_iota = lax.iota(jnp.int32, LANES)
    bufs = (buf0, buf1)
    sems = (sem0, sem1)

    copies = [None] * ROWS_PER_WORKER
    copies[0] = pltpu.async_copy(x_hbm.at[base_row], buf0, sem0)
    res = jnp.zeros((LANES,), jnp.int32)
    for j in range(ROWS_PER_WORKER):
        copies[j].wait()
        if j + 1 < ROWS_PER_WORKER:
            copies[j + 1] = pltpu.async_copy(
                x_hbm.at[base_row + j + 1], bufs[(j + 1) % 2], sems[(j + 1) % 2]
            )
        val = _row_argmin(bufs[j % 2], lane_iota)
        res = jnp.where(lane_iota == j, val, res)
    res_ref[...] = res
    pltpu.sync_copy(res_ref, shared.at[s])
    plsc.subcore_barrier()

    @pl.when(s == 0)
    def _():
        pltpu.sync_copy(shared, gbuf)
        for v in range(ROWS_PER_CORE // LANES):
            rp = lane_iota + v * LANES
            g = plsc.load_gather(gbuf, [rp >> 2, rp & 3])
            obuf[pl.ds(v * LANES, LANES)] = g
        pltpu.sync_copy(obuf, out_flat.at[pl.ds(c * ROWS_PER_CORE, ROWS_PER_CORE)])


def kernel(x):
    out_flat = _argmin_sc(x)
    return (out_flat.reshape(ROWS, 1), out_flat)


# speedup vs baseline: 1.5246x; 1.0048x over previous
import functools

import jax
import jax.numpy as jnp
from jax import lax
from jax.experimental import pallas as pl
from jax.experimental.pallas import tpu as pltpu
from jax.experimental.pallas import tpu_sc as plsc

ROWS = 128
COLS = 32768
LANES = 16
NUM_CORES = 2
NUM_SUBCORES = 16
NUM_WORKERS = NUM_CORES * NUM_SUBCORES
SC_ROWS = NUM_WORKERS
TC_ROWS = ROWS - SC_ROWS
VECS = COLS // LANES
UNROLL = 16
NACC = 4
TC_BLOCK_ROWS = 8


def _row_argmin(buf, lane_iota):
    def body(i, carry):
        minvs, minis = carry
        minvs = list(minvs)
        minis = list(minis)
        base = i * (LANES * UNROLL)
        for u in range(UNROLL):
            k = u % NACC
            off = base + u * LANES
            v = buf[pl.ds(off, LANES)]
            idxv = lane_iota + off
            pred = v < minvs[k]
            minvs[k] = jnp.where(pred, v, minvs[k])
            minis[k] = jnp.where(pred, idxv, minis[k])
        return tuple(minvs), tuple(minis)

    minv0 = jnp.full((LANES,), jnp.inf, jnp.float32)
    mini0 = jnp.zeros((LANES,), jnp.int32)
    minvs, minis = lax.fori_loop(
        0, VECS // UNROLL, body, ((minv0,) * NACC, (mini0,) * NACC)
    )
    minv, mini = minvs[0], minis[0]
    for k in range(1, NACC):
        pred = (minvs[k] < minv) | ((minvs[k] == minv) & (minis[k] < mini))
        minv = jnp.where(pred, minvs[k], minv)
        mini = jnp.where(pred, minis[k], mini)
    m = jnp.min(minv)
    cand = jnp.where(minv == m, mini, jnp.int32(COLS))
    return jnp.min(cand)


@functools.partial(
    pl.kernel,
    out_type=jax.ShapeDtypeStruct((NUM_WORKERS, LANES), jnp.int32),
    mesh=plsc.VectorSubcoreMesh(
        core_axis_name="c",
        subcore_axis_name="s",
        num_cores=NUM_CORES,
        num_subcores=NUM_SUBCORES,
    ),
    scratch_types=[
        pltpu.VMEM((COLS,), jnp.float32),
        pltpu.VMEM((LANES,), jnp.int32),
    ],
    compiler_params=pltpu.CompilerParams(needs_layout_passes=False),
)
def _argmin_sc(x_hbm, out_hbm, buf, res_ref):
    wid = lax.axis_index("s") * NUM_CORES + lax.axis_index("c")
    lane_iota = lax.iota(jnp.int32, LANES)
    pltpu.sync_copy(x_hbm.at[wid], buf)
    val = _row_argmin(buf, lane_iota)
    res_ref[...] = jnp.where(lane_iota == 0, val, jnp.int32(0))
    pltpu.sync_copy(res_ref, out_hbm.at[wid])


def _argmin_tc_body(x_ref, out_ref):
    xb = x_ref[...]
    rm = jnp.min(xb, axis=1, keepdims=True)
    idx = lax.broadcasted_iota(jnp.int32, (TC_BLOCK_ROWS, COLS), 1)
    cand = jnp.where(xb == rm, idx, jnp.int32(COLS))
    out_ref[...] = jnp.min(cand, axis=1).reshape(1, 1, TC_BLOCK_ROWS)


_argmin_tc = pl.pallas_call(
    _argmin_tc_body,
    grid=(TC_ROWS // TC_BLOCK_ROWS,),
    in_specs=[
        pl.BlockSpec(
            (TC_BLOCK_ROWS, COLS), lambda i: (i + SC_ROWS // TC_BLOCK_ROWS, 0)
        )
    ],
    out_specs=pl.BlockSpec((1, 1, TC_BLOCK_ROWS), lambda i: (i, 0, 0)),
    out_shape=jax.ShapeDtypeStruct(
        (TC_ROWS // TC_BLOCK_ROWS, 1, TC_BLOCK_ROWS), jnp.int32
    ),
)


def kernel(x):
    sc_pad = _argmin_sc(x)
    tc_out = _argmin_tc(x)
    flat = jnp.concatenate([sc_pad[:, 0], tc_out.reshape(TC_ROWS)])
    return (flat.reshape(ROWS, 1), flat)


# speedup vs baseline: 1.6497x; 1.0821x over previous
import functools

import jax
import jax.numpy as jnp
from jax import lax
from jax.experimental import pallas as pl
from jax.experimental.pallas import tpu as pltpu
from jax.experimental.pallas import tpu_sc as plsc

ROWS = 128
COLS = 32768
LANES = 16
NUM_CORES = 2
NUM_SUBCORES = 16
NUM_WORKERS = NUM_CORES * NUM_SUBCORES
SC_ROWS_PER_WORKER = 2
SC_ROWS = NUM_WORKERS * SC_ROWS_PER_WORKER
TC_ROWS = ROWS - SC_ROWS
VECS = COLS // LANES
UNROLL = 16
NACC = 4
TC_BLOCK_ROWS = 8


def _row_argmin(buf, lane_iota):
    def body(i, carry):
        minvs, minis = carry
        minvs = list(minvs)
        minis = list(minis)
        base = i * (LANES * UNROLL)
        for u in range(UNROLL):
            k = u % NACC
            off = base + u * LANES
            v = buf[pl.ds(off, LANES)]
            idxv = lane_iota + off
            pred = v < minvs[k]
            minvs[k] = jnp.where(pred, v, minvs[k])
            minis[k] = jnp.where(pred, idxv, minis[k])
        return tuple(minvs), tuple(minis)

    minv0 = jnp.full((LANES,), jnp.inf, jnp.float32)
    mini0 = jnp.zeros((LANES,), jnp.int32)
    minvs, minis = lax.fori_loop(
        0, VECS // UNROLL, body, ((minv0,) * NACC, (mini0,) * NACC)
    )
    minv, mini = minvs[0], minis[0]
    for k in range(1, NACC):
        pred = (minvs[k] < minv) | ((minvs[k] == minv) & (minis[k] < mini))
        minv = jnp.where(pred, minvs[k], minv)
        mini = jnp.where(pred, minis[k], mini)
    m = jnp.min(minv)
    cand = jnp.where(minv == m, mini, jnp.int32(COLS))
    return jnp.min(cand)


@functools.partial(
    pl.kernel,
    out_type=jax.ShapeDtypeStruct((NUM_WORKERS, LANES), jnp.int32),
    mesh=plsc.VectorSubcoreMesh(
        core_axis_name="c",
        subcore_axis_name="s",
        num_cores=NUM_CORES,
        num_subcores=NUM_SUBCORES,
    ),
    scratch_types=[
        pltpu.VMEM((COLS,), jnp.float32),
        pltpu.VMEM((COLS,), jnp.float32),
        pltpu.VMEM((LANES,), jnp.int32),
        pltpu.SemaphoreType.DMA,
        pltpu.SemaphoreType.DMA,
    ],
    compiler_params=pltpu.CompilerParams(needs_layout_passes=False),
)
def _argmin_sc(x_hbm, out_hbm, buf0, buf1, res_ref, sem0, sem1):
    wid = lax.axis_index("s") * NUM_CORES + lax.axis_index("c")
    base = wid * SC_ROWS_PER_WORKER
    lane_iota = lax.iota(jnp.int32, LANES)
    bufs = (buf0, buf1)
    sems = (sem0, sem1)
    copies = [None] * SC_ROWS_PER_WORKER
    copies[0] = pltpu.async_copy(x_hbm.at[base], buf0, sem0)
    res = jnp.zeros((LANES,), jnp.int32)
    for j in range(SC_ROWS_PER_WORKER):
        copies[j].wait()
        if j + 1 < SC_ROWS_PER_WORKER:
            copies[j + 1] = pltpu.async_copy(
                x_hbm.at[base + j + 1], bufs[(j + 1) % 2], sems[(j + 1) % 2]
            )
        val = _row_argmin(bufs[j % 2], lane_iota)
        res = jnp.where(lane_iota == j, val, res)
    res_ref[...] = res
    pltpu.sync_copy(res_ref, out_hbm.at[wid])


def _argmin_tc_body(x_ref, out_ref):
    xb = x_ref[...]
    rm = jnp.min(xb, axis=1, keepdims=True)
    idx = lax.broadcasted_iota(jnp.int32, (TC_BLOCK_ROWS, COLS), 1)
    cand = jnp.where(xb == rm, idx, jnp.int32(COLS))
    out_ref[...] = jnp.min(cand, axis=1).reshape(1, 1, TC_BLOCK_ROWS)


_argmin_tc = pl.pallas_call(
    _argmin_tc_body,
    grid=(TC_ROWS // TC_BLOCK_ROWS,),
    in_specs=[
        pl.BlockSpec(
            (TC_BLOCK_ROWS, COLS), lambda i: (i + SC_ROWS // TC_BLOCK_ROWS, 0)
        )
    ],
    out_specs=pl.BlockSpec((1, 1, TC_BLOCK_ROWS), lambda i: (i, 0, 0)),
    out_shape=jax.ShapeDtypeStruct(
        (TC_ROWS // TC_BLOCK_ROWS, 1, TC_BLOCK_ROWS), jnp.int32
    ),
)


def kernel(x):
    sc_pad = _argmin_sc(x)
    tc_out = _argmin_tc(x)
    flat = jnp.concatenate(
        [sc_pad[:, :SC_ROWS_PER_WORKER].reshape(SC_ROWS), tc_out.reshape(TC_ROWS)]
    )
    return (flat.reshape(ROWS, 1), flat)


# speedup vs baseline: 1.7132x; 1.0385x over previous
import functools

import jax
import jax.numpy as jnp
from jax import lax
from jax.experimental import pallas as pl
from jax.experimental.pallas import tpu as pltpu
from jax.experimental.pallas import tpu_sc as plsc

ROWS = 128
COLS = 32768
LANES = 16
NUM_CORES = 2
NUM_SUBCORES = 16
NUM_WORKERS = NUM_CORES * NUM_SUBCORES
SC_ROWS_PER_WORKER = 2
SC_ROWS = NUM_WORKERS * SC_ROWS_PER_WORKER
TC_ROWS = ROWS - SC_ROWS
VECS = COLS // LANES
UNROLL = 16
NACC = 4
TC_BLOCK_ROWS = 16


def _row_argmin(buf, lane_iota):
    def body(i, carry):
        minvs, minis = carry
        minvs = list(minvs)
        minis = list(minis)
        base = i * (LANES * UNROLL)
        for u in range(UNROLL):
            k = u % NACC
            off = base + u * LANES
            v = buf[pl.ds(off, LANES)]
            idxv = lane_iota + off
            pred = v < minvs[k]
            minvs[k] = jnp.where(pred, v, minvs[k])
            minis[k] = jnp.where(pred, idxv, minis[k])
        return tuple(minvs), tuple(minis)

    minv0 = jnp.full((LANES,), jnp.inf, jnp.float32)
    mini0 = jnp.zeros((LANES,), jnp.int32)
    minvs, minis = lax.fori_loop(
        0, VECS // UNROLL, body, ((minv0,) * NACC, (mini0,) * NACC)
    )
    minv, mini = minvs[0], minis[0]
    for k in range(1, NACC):
        pred = (minvs[k] < minv) | ((minvs[k] == minv) & (minis[k] < mini))
        minv = jnp.where(pred, minvs[k], minv)
        mini = jnp.where(pred, minis[k], mini)
    m = jnp.min(minv)
    cand = jnp.where(minv == m, mini, jnp.int32(COLS))
    return jnp.min(cand)


@functools.partial(
    pl.kernel,
    out_type=jax.ShapeDtypeStruct((SC_ROWS, LANES), jnp.int32),
    mesh=plsc.VectorSubcoreMesh(
        core_axis_name="c",
        subcore_axis_name="s",
        num_cores=NUM_CORES,
        num_subcores=NUM_SUBCORES,
    ),
    scratch_types=[
        pltpu.VMEM((COLS,), jnp.float32),
        pltpu.VMEM((COLS,), jnp.float32),
        pltpu.VMEM((LANES,), jnp.int32),
        pltpu.SemaphoreType.DMA,
        pltpu.SemaphoreType.DMA,
    ],
    compiler_params=pltpu.CompilerParams(needs_layout_passes=False),
)
def _argmin_sc(x_hbm, out_hbm, buf0, buf1, res_ref, sem0, sem1):
    wid = lax.axis_index("s") * NUM_CORES + lax.axis_index("c")
    base = wid * SC_ROWS_PER_WORKER
    lane_iota = lax.iota(jnp.int32, LANES)
    bufs = (buf0, buf1)
    sems = (sem0, sem1)
    copies = [None] * SC_ROWS_PER_WORKER
    copies[0] = pltpu.async_copy(x_hbm.at[base], buf0, sem0)
    res = jnp.zeros((LANES,), jnp.int32)
    for j in range(SC_ROWS_PER_WORKER):
        copies[j].wait()
        if j + 1 < SC_ROWS_PER_WORKER:
            copies[j + 1] = pltpu.async_copy(
                x_hbm.at[base + j + 1], bufs[(j + 1) % 2], sems[(j + 1) % 2]
            )
        val = _row_argmin(bufs[j % 2], lane_iota)
        res_ref[...] = jnp.where(lane_iota == 0, val, res)
        pltpu.sync_copy(res_ref, out_hbm.at[base + j])


def _argmin_tc_body(x_ref, out_ref):
    xb = x_ref[...]
    rm = jnp.min(xb, axis=1, keepdims=True)
    idx = lax.broadcasted_iota(jnp.int32, (TC_BLOCK_ROWS, COLS), 1)
    cand = jnp.where(xb == rm, idx, jnp.int32(COLS))
    out_ref[...] = jnp.min(cand, axis=1).reshape(1, 1, TC_BLOCK_ROWS)


_argmin_tc = pl.pallas_call(
    _argmin_tc_body,
    grid=(TC_ROWS // TC_BLOCK_ROWS,),
    in_specs=[
        pl.BlockSpec(
            (TC_BLOCK_ROWS, COLS), lambda i: (i + SC_ROWS // TC_BLOCK_ROWS, 0)
        )
    ],
    out_specs=pl.BlockSpec((1, 1, TC_BLOCK_ROWS), lambda i: (i, 0, 0)),
    out_shape=jax.ShapeDtypeStruct(
        (TC_ROWS // TC_BLOCK_ROWS, 1, TC_BLOCK_ROWS), jnp.int32
    ),
)


def kernel(x):
    tc_out = _argmin_tc(x)
    sc_pad = _argmin_sc(x)
    flat = jnp.concatenate([sc_pad[:, 0], tc_out.reshape(TC_ROWS)])
    return (flat.reshape(ROWS, 1), flat)


# speedup vs baseline: 1.7830x; 1.0408x over previous
import functools

import jax
import jax.numpy as jnp
from jax import lax
from jax.experimental import pallas as pl
from jax.experimental.pallas import tpu as pltpu
from jax.experimental.pallas import tpu_sc as plsc

ROWS = 128
COLS = 32768
LANES = 16
NUM_CORES = 2
NUM_SUBCORES = 16
NUM_WORKERS = NUM_CORES * NUM_SUBCORES
SC_ROWS_PER_WORKER = 1
SC_ROWS = NUM_WORKERS * SC_ROWS_PER_WORKER
TC_ROWS = ROWS - SC_ROWS
VECS = COLS // LANES
UNROLL = 16
NACC = 4
TC_BLOCK_ROWS = 16


def _row_argmin(buf, lane_iota):
    def body(i, carry):
        minvs, minis = carry
        minvs = list(minvs)
        minis = list(minis)
        base = i * (LANES * UNROLL)
        for u in range(UNROLL):
            k = u % NACC
            off = base + u * LANES
            v = buf[pl.ds(off, LANES)]
            idxv = lane_iota + off
            pred = v < minvs[k]
            minvs[k] = jnp.where(pred, v, minvs[k])
            minis[k] = jnp.where(pred, idxv, minis[k])
        return tuple(minvs), tuple(minis)

    minv0 = jnp.full((LANES,), jnp.inf, jnp.float32)
    mini0 = jnp.zeros((LANES,), jnp.int32)
    minvs, minis = lax.fori_loop(
        0, VECS // UNROLL, body, ((minv0,) * NACC, (mini0,) * NACC)
    )
    minv, mini = minvs[0], minis[0]
    for k in range(1, NACC):
        pred = (minvs[k] < minv) | ((minvs[k] == minv) & (minis[k] < mini))
        minv = jnp.where(pred, minvs[k], minv)
        mini = jnp.where(pred, minis[k], mini)
    m = jnp.min(minv)
    cand = jnp.where(minv == m, mini, jnp.int32(COLS))
    return jnp.min(cand)


@functools.partial(
    pl.kernel,
    out_type=jax.ShapeDtypeStruct((SC_ROWS, LANES), jnp.int32),
    mesh=plsc.VectorSubcoreMesh(
        core_axis_name="c",
        subcore_axis_name="s",
        num_cores=NUM_CORES,
        num_subcores=NUM_SUBCORES,
    ),
    scratch_types=[
        pltpu.VMEM((COLS,), jnp.float32),
        pltpu.VMEM((COLS,), jnp.float32),
        pltpu.VMEM((LANES,), jnp.int32),
        pltpu.SemaphoreType.DMA,
        pltpu.SemaphoreType.DMA,
    ],
    compiler_params=pltpu.CompilerParams(needs_layout_passes=False),
)
def _argmin_sc(x_hbm, out_hbm, buf0, buf1, res_ref, sem0, sem1):
    wid = lax.axis_index("s") * NUM_CORES + lax.axis_index("c")
    base = wid * SC_ROWS_PER_WORKER
    lane_iota = lax.iota(jnp.int32, LANES)
    bufs = (buf0, buf1)
    sems = (sem0, sem1)
    copies = [None] * SC_ROWS_PER_WORKER
    copies[0] = pltpu.async_copy(x_hbm.at[base], buf0, sem0)
    res = jnp.zeros((LANES,), jnp.int32)
    for j in range(SC_ROWS_PER_WORKER):
        copies[j].wait()
        if j + 1 < SC_ROWS_PER_WORKER:
            copies[j + 1] = pltpu.async_copy(
                x_hbm.at[base + j + 1], bufs[(j + 1) % 2], sems[(j + 1) % 2]
            )
        val = _row_argmin(bufs[j % 2], lane_iota)
        res_ref[...] = jnp.where(lane_iota == 0, val, res)
        pltpu.sync_copy(res_ref, out_hbm.at[base + j])


def _argmin_tc_body(x_ref, out_ref):
    xb = x_ref[...]
    rm = jnp.min(xb, axis=1, keepdims=True)
    idx = lax.broadcasted_iota(jnp.int32, (TC_BLOCK_ROWS, COLS), 1)
    cand = jnp.where(xb == rm, idx, jnp.int32(COLS))
    out_ref[...] = jnp.min(cand, axis=1).reshape(1, 1, TC_BLOCK_ROWS)


_argmin_tc = pl.pallas_call(
    _argmin_tc_body,
    grid=(TC_ROWS // TC_BLOCK_ROWS,),
    in_specs=[
        pl.BlockSpec(
            (TC_BLOCK_ROWS, COLS), lambda i: (i + SC_ROWS // TC_BLOCK_ROWS, 0)
        )
    ],
    out_specs=pl.BlockSpec((1, 1, TC_BLOCK_ROWS), lambda i: (i, 0, 0)),
    out_shape=jax.ShapeDtypeStruct(
        (TC_ROWS // TC_BLOCK_ROWS, 1, TC_BLOCK_ROWS), jnp.int32
    ),
)


def kernel(x):
    tc_out = _argmin_tc(x)
    sc_pad = _argmin_sc(x)
    flat = jnp.concatenate([sc_pad[:, 0], tc_out.reshape(TC_ROWS)])
    return (flat.reshape(ROWS, 1), flat)
